# Initial kernel scaffold; baseline (speedup 1.0000x reference)
import jax
import jax.numpy as jnp
from jax.experimental import pallas as pl


def kernel(x, edge_index, W1, b1, Wmu, bmu, Wls, bls):
    raise NotImplementedError("write your pallas kernel here")


# speedup vs baseline: 33.9319x; 33.9319x over previous
import functools

import jax
import jax.numpy as jnp
from jax import lax
from jax.experimental import pallas as pl
from jax.experimental.pallas import tpu as pltpu
from jax.experimental.pallas import tpu_sc as plsc

N = 10000
E = 320000
CH = 128
HALF = 64
DW = 16

NC = 2
NS = 16
CHUNK = 128
CPT = 160
NCHUNK = NS * CPT
EPAD = NCHUNK * CHUNK
K = 2
NG = CPT // K
DCPT = NCHUNK // (NC * NS)

ACCROWS = 10240
RPT = ACCROWS // NS


def _sc_deg_body(dstc, zeros16, ones16, out, acc, dst_v, ones_v):
    c = lax.axis_index("c")
    s = lax.axis_index("s")
    r0 = s * RPT
    pltpu.sync_copy(zeros16.at[pl.ds(r0, RPT)], acc.at[pl.ds(r0, RPT)])
    pltpu.sync_copy(ones16, ones_v)
    base = c * (NCHUNK // NC) + s * DCPT
    pltpu.sync_copy(dstc.at[pl.ds(base, DCPT)], dst_v)
    plsc.subcore_barrier()

    def body(k, carry):
        pltpu.sync_copy(ones_v, acc.at[dst_v.at[k]], add=True)
        return carry

    lax.fori_loop(0, DCPT, body, 0)
    plsc.subcore_barrier()
    pltpu.sync_copy(acc.at[pl.ds(r0, RPT)], out.at[c].at[pl.ds(r0, RPT)])


def _sc_conv_body(mflat, srcc, dstc, zeros, out, acc, src_v, dst_v, bufs,
                  sem0, sem1):
    c = lax.axis_index("c")
    s = lax.axis_index("s")
    r0 = s * RPT
    pltpu.sync_copy(zeros.at[pl.ds(r0, RPT)], acc.at[pl.ds(r0, RPT)])
    pltpu.sync_copy(srcc.at[c].at[pl.ds(s * CPT, CPT)], src_v)
    pltpu.sync_copy(dstc.at[pl.ds(s * CPT, CPT)], dst_v)
    plsc.subcore_barrier()

    sems = (sem0, sem1)

    def start_group(g, par):
        for b in range(K):
            pltpu.async_copy(mflat.at[src_v.at[g * K + b]], bufs.at[par, b],
                             sems[par])

    def drain_scatter_group(g, par):
        for b in range(K):
            pltpu.make_async_copy(mflat.at[pl.ds(0, CHUNK)], bufs.at[par, b],
                                  sems[par]).wait()
        for b in range(K):
            pltpu.sync_copy(bufs.at[par, b], acc.at[dst_v.at[g * K + b]],
                            add=True)

    start_group(0, 0)
    start_group(1, 1)

    def body(gg, carry):
        g0 = gg * 2
        for par in (0, 1):
            g = g0 + par
            drain_scatter_group(g, par)

            @pl.when(g + 2 < NG)
            def _():
                start_group(g + 2, par)

        return carry

    lax.fori_loop(0, NG // 2, body, 0)
    plsc.subcore_barrier()
    pltpu.sync_copy(acc.at[pl.ds(r0, RPT)], out.at[c].at[pl.ds(r0, RPT)])


def _dis_from_degp(degp_ref):
    deg = degp_ref[0, :, 0:1] + degp_ref[1, :, 0:1] + 1.0
    return lax.rsqrt(deg)[:N, :]


def _tc_m1_body(x_ref, w1_ref, degp_ref, out_ref):
    dis = _dis_from_degp(degp_ref)
    h = jnp.dot(x_ref[...], w1_ref[...], preferred_element_type=jnp.float32)
    m = h * dis
    out_ref[0] = m[:, :HALF]
    out_ref[1] = m[:, HALF:]


def _tc_m2_body(acc1_ref, m1_ref, degp_ref, b1_ref, wcat_ref, out_ref):
    dis = _dis_from_degp(degp_ref)
    a = jnp.concatenate(
        [acc1_ref[0, :N, :] + m1_ref[0], acc1_ref[1, :N, :] + m1_ref[1]],
        axis=1)
    h = jnp.maximum(dis * a + b1_ref[...][None, :], 0.0)
    m2 = jnp.dot(h, wcat_ref[...], preferred_element_type=jnp.float32) * dis
    out_ref[0] = m2[:, :HALF]
    out_ref[1] = m2[:, HALF:]


def _tc_out_body(acc2_ref, m2_ref, degp_ref, bmu_ref, bls_ref, mu_ref,
                 ls_ref):
    dis = _dis_from_degp(degp_ref)
    mu_ref[...] = dis * (acc2_ref[0, :N, :] + m2_ref[0]) + bmu_ref[...][None, :]
    ls_ref[...] = dis * (acc2_ref[1, :N, :] + m2_ref[1]) + bls_ref[...][None, :]


def _make_sc_kernels():
    mesh = plsc.VectorSubcoreMesh(core_axis_name="c", subcore_axis_name="s")
    params = pltpu.CompilerParams(use_tc_tiling_on_sc=False)
    deg_kernel = functools.partial(
        pl.kernel,
        out_type=jax.ShapeDtypeStruct((NC, ACCROWS, DW), jnp.float32),
        mesh=mesh,
        compiler_params=params,
        scratch_types=[
            pltpu.VMEM_SHARED((ACCROWS, DW), jnp.float32),
            pltpu.VMEM((DCPT, CHUNK), jnp.int32),
            pltpu.VMEM((CHUNK, DW), jnp.float32),
        ],
    )(_sc_deg_body)
    conv_kernel = functools.partial(
        pl.kernel,
        out_type=jax.ShapeDtypeStruct((NC, ACCROWS, HALF), jnp.float32),
        mesh=mesh,
        compiler_params=params,
        scratch_types=[
            pltpu.VMEM_SHARED((ACCROWS, HALF), jnp.float32),
            pltpu.VMEM((CPT, CHUNK), jnp.int32),
            pltpu.VMEM((CPT, CHUNK), jnp.int32),
            pltpu.VMEM((2, K, CHUNK, HALF), jnp.float32),
            pltpu.SemaphoreType.DMA,
            pltpu.SemaphoreType.DMA,
        ],
    )(_sc_conv_body)
    return deg_kernel, conv_kernel


def kernel(x, edge_index, W1, b1, Wmu, bmu, Wls, bls):
    src = edge_index[0].astype(jnp.int32)
    dst = edge_index[1].astype(jnp.int32)
    npad = EPAD - E
    ar = jnp.arange(npad, dtype=jnp.int32)
    src_p = jnp.concatenate([src, ar % N])
    dst_p = jnp.concatenate([dst, N + ar % (ACCROWS - N)])
    dstc = dst_p.reshape(NCHUNK, CHUNK)
    srcc = jnp.stack([src_p, src_p + N]).reshape(NC, NCHUNK, CHUNK)
    zeros64 = jnp.zeros((ACCROWS, HALF), jnp.float32)
    zeros16 = jnp.zeros((ACCROWS, DW), jnp.float32)
    ones16 = jnp.ones((CHUNK, DW), jnp.float32)

    deg_kernel, conv_kernel = _make_sc_kernels()

    degp = deg_kernel(dstc, zeros16, ones16)

    m1 = pl.pallas_call(
        _tc_m1_body,
        out_shape=jax.ShapeDtypeStruct((NC, N, HALF), jnp.float32),
    )(x, W1, degp)

    acc1 = conv_kernel(m1.reshape(NC * N, HALF), srcc, dstc, zeros64)

    wcat = jnp.concatenate([Wmu, Wls], axis=1)
    m2 = pl.pallas_call(
        _tc_m2_body,
        out_shape=jax.ShapeDtypeStruct((NC, N, HALF), jnp.float32),
    )(acc1, m1, degp, b1, wcat)

    acc2 = conv_kernel(m2.reshape(NC * N, HALF), srcc, dstc, zeros64)

    mu, logstd = pl.pallas_call(
        _tc_out_body,
        out_shape=(jax.ShapeDtypeStruct((N, HALF), jnp.float32),
                   jax.ShapeDtypeStruct((N, HALF), jnp.float32)),
    )(acc2, m2, degp, bmu, bls)
    return (mu, logstd)
